# Initial kernel scaffold; baseline (speedup 1.0000x reference)
import jax
import jax.numpy as jnp
from jax.experimental import pallas as pl


def kernel(x, edge_index, params):
    raise NotImplementedError("write your pallas kernel here")


# speedup vs baseline: 19.1296x; 19.1296x over previous
import functools

import jax
import jax.numpy as jnp
from jax import lax
from jax.experimental import pallas as pl
from jax.experimental.pallas import tpu as pltpu
from jax.experimental.pallas import tpu_sc as plsc

N = 10000
E = 320000
INPUT_LENGTH = 24

NC = 2
NS = 16
CH = 162
EW = CH * 128
EP = NS * EW
NPAD = 10240
NACC = 10240
ROWS_PER_SUB = NACC // NS

NDR = NPAD // 128


def _make_edge_kernel(F):
    FH = F // 2
    NQ = FH // 16
    mesh = plsc.VectorSubcoreMesh(core_axis_name="c", subcore_axis_name="s")

    @functools.partial(
        pl.kernel,
        out_type=(jax.ShapeDtypeStruct((NC, NACC, FH), jnp.float32),
                  jax.ShapeDtypeStruct((NC, NDR, 128), jnp.float32)),
        mesh=mesh,
        compiler_params=pltpu.CompilerParams(needs_layout_passes=False,
                                             use_tc_tiling_on_sc=False),
        scratch_types=[
            pltpu.VMEM((CH, 128), jnp.int32),
            pltpu.VMEM((CH, 128), jnp.int32),
            pltpu.VMEM((NDR, 128), jnp.float32),
            pltpu.VMEM((NDR, 128), jnp.float32),
            pltpu.VMEM((8, 16), jnp.float32),
            pltpu.VMEM((128, FH), jnp.float32),
            pltpu.VMEM((NDR, 128), jnp.float32),
            pltpu.VMEM((1, NDR), jnp.int32),
            pltpu.VMEM_SHARED((NACC, FH), jnp.float32),
            pltpu.VMEM_SHARED((NDR, 128), jnp.float32),
            pltpu.SemaphoreType.DMA,
        ],
    )
    def edge_kernel(ha_hbm, hb_hbm, as_hbm, ad_hbm, src_hbm, dst_hbm, out_hbm,
                    oden_hbm, src_v, dst_v, as_v, ad_v, ea_v, rows_v,
                    denl_v, iden_v, acc_sh, den_sh, sem):
        c = lax.axis_index("c")
        s = lax.axis_index("s")

        pltpu.sync_copy(src_hbm.at[s], src_v)
        pltpu.sync_copy(dst_hbm.at[s], dst_v)
        pltpu.sync_copy(as_hbm, as_v)
        pltpu.sync_copy(ad_hbm, ad_v)

        zv = jnp.zeros((16,), jnp.float32)

        def zden_body(r, carry):
            for q in range(8):
                denl_v[r, pl.ds(q * 16, 16)] = zv
            return carry

        lax.fori_loop(0, NDR, zden_body, 0)

        def zrow_body(r, carry):
            for q in range(NQ):
                rows_v[r, pl.ds(q * 16, 16)] = zv
            return carry

        lax.fori_loop(0, 128, zrow_body, 0)

        for k in range(ROWS_PER_SUB // 128):
            pltpu.sync_copy(
                rows_v, acc_sh.at[pl.ds(s * ROWS_PER_SUB + k * 128, 128)])

        @pl.when(s < NDR // 8)
        def _():
            pltpu.sync_copy(denl_v.at[pl.ds(0, 8)],
                            den_sh.at[pl.ds(s * 8, 8)])

        for i in range(NDR // 16):
            iden_v[0, pl.ds(i * 16, 16)] = (
                lax.iota(jnp.int32, 16) + jnp.int32(i * 16))
        plsc.subcore_barrier()

        def chunk_body(j, carry):
            @pl.when(c == 0)
            def _():
                pltpu.async_copy(ha_hbm.at[src_v.at[j]], rows_v, sem).wait()

            @pl.when(c == 1)
            def _():
                pltpu.async_copy(hb_hbm.at[src_v.at[j]], rows_v, sem).wait()

            for l in range(8):
                sv = src_v[j, pl.ds(l * 16, 16)]
                dv = dst_v[j, pl.ds(l * 16, 16)]
                a = (plsc.load_gather(as_v, [sv >> 7, sv & 127])
                     + plsc.load_gather(ad_v, [dv >> 7, dv & 127]))
                a = jnp.maximum(a, 0.2 * a)
                ea = jnp.exp(a)
                ea_v[l] = ea
                plsc.addupdate_scatter(denl_v, [dv >> 7, dv & 127], ea)

            def row_body(r, carry2):
                rv = jnp.full((16,), r, jnp.int32)
                eav = plsc.load_gather(ea_v, [rv >> 4, rv & 15])
                for q in range(NQ):
                    rows_v[r, pl.ds(q * 16, 16)] = (
                        rows_v[r, pl.ds(q * 16, 16)] * eav)
                return carry2

            lax.fori_loop(0, 128, row_body, 0)
            pltpu.sync_copy(rows_v, acc_sh.at[dst_v.at[j]], add=True)
            return carry

        lax.fori_loop(0, CH, chunk_body, 0)

        pltpu.sync_copy(denl_v, den_sh.at[iden_v.at[0]], add=True)
        plsc.subcore_barrier()
        pltpu.sync_copy(acc_sh.at[pl.ds(s * ROWS_PER_SUB, ROWS_PER_SUB)],
                        out_hbm.at[c, pl.ds(s * ROWS_PER_SUB, ROWS_PER_SUB)])

        @pl.when(s < NDR // 8)
        def _():
            pltpu.sync_copy(den_sh.at[pl.ds(s * 8, 8)],
                            oden_hbm.at[c, pl.ds(s * 8, 8)])

    return edge_kernel


_edge128 = _make_edge_kernel(128)
_edge64 = _make_edge_kernel(64)


_R = 1000


def _leaky(x, slope):
    return jnp.maximum(x, slope * x)


def _wspec(shape):
    return pl.BlockSpec(shape, lambda i: (0,) * len(shape))


def _pre_body(xs_ref, w1t_ref, b1_ref, w2t_ref, b2_ref, gwt_ref, asw_ref,
              adw_ref, ha_ref, hb_ref, as_ref, ad_ref):
    s1 = jax.nn.sigmoid(jnp.dot(xs_ref[...], w1t_ref[...],
                                preferred_element_type=jnp.float32) + b1_ref[...])
    s2 = jax.nn.sigmoid(jnp.dot(s1, w2t_ref[...],
                                preferred_element_type=jnp.float32) + b2_ref[...])
    h = jnp.dot(s2, gwt_ref[...], preferred_element_type=jnp.float32)
    ha_ref[...] = h[:, :64]
    hb_ref[...] = h[:, 64:]
    as_ref[...] = (h * asw_ref[...]).sum(-1, keepdims=True)
    ad_ref[...] = (h * adw_ref[...]).sum(-1, keepdims=True)


def _pre_stage(xs, w1t, b1, w2t, b2, gwt, asw, adw):
    grid = (N // _R,)
    return pl.pallas_call(
        _pre_body,
        grid=grid,
        in_specs=[
            pl.BlockSpec((_R, 40), lambda i: (i, 0)),
            _wspec((40, 128)), _wspec((1, 128)),
            _wspec((128, 128)), _wspec((1, 128)),
            _wspec((128, 128)), _wspec((1, 128)), _wspec((1, 128)),
        ],
        out_specs=[
            pl.BlockSpec((_R, 64), lambda i: (i, 0)),
            pl.BlockSpec((_R, 64), lambda i: (i, 0)),
            pl.BlockSpec((_R, 1), lambda i: (i, 0)),
            pl.BlockSpec((_R, 1), lambda i: (i, 0)),
        ],
        out_shape=[
            jax.ShapeDtypeStruct((N, 64), jnp.float32),
            jax.ShapeDtypeStruct((N, 64), jnp.float32),
            jax.ShapeDtypeStruct((N, 1), jnp.float32),
            jax.ShapeDtypeStruct((N, 1), jnp.float32),
        ],
    )(xs, w1t, b1, w2t, b2, gwt, asw, adw)


def _make_mid_body(fh_in, f_out):
    def _mid_body(num_ref, den_ref, b_ref, gwt_ref, asw_ref, adw_ref,
                  ha_ref, hb_ref, as_ref, ad_ref):
        ns = jnp.concatenate([num_ref[0], num_ref[1]], axis=-1)
        den = den_ref[...]
        hp = _leaky(ns / den + b_ref[...], 0.01)
        h = jnp.dot(hp, gwt_ref[...], preferred_element_type=jnp.float32)
        ha_ref[...] = h[:, :f_out // 2]
        hb_ref[...] = h[:, f_out // 2:]
        as_ref[...] = (h * asw_ref[...]).sum(-1, keepdims=True)
        ad_ref[...] = (h * adw_ref[...]).sum(-1, keepdims=True)
    return _mid_body


def _mid_stage(num, den, b, gwt, asw, adw, f_in, f_out):
    grid = (N // _R,)
    fh_in = f_in // 2
    fh_out = f_out // 2
    return pl.pallas_call(
        _make_mid_body(fh_in, f_out),
        grid=grid,
        in_specs=[
            pl.BlockSpec((2, _R, fh_in), lambda i: (0, i, 0)),
            pl.BlockSpec((_R, 1), lambda i: (i, 0)),
            _wspec((1, f_in)),
            _wspec((f_in, f_out)), _wspec((1, f_out)), _wspec((1, f_out)),
        ],
        out_specs=[
            pl.BlockSpec((_R, fh_out), lambda i: (i, 0)),
            pl.BlockSpec((_R, fh_out), lambda i: (i, 0)),
            pl.BlockSpec((_R, 1), lambda i: (i, 0)),
            pl.BlockSpec((_R, 1), lambda i: (i, 0)),
        ],
        out_shape=[
            jax.ShapeDtypeStruct((N, fh_out), jnp.float32),
            jax.ShapeDtypeStruct((N, fh_out), jnp.float32),
            jax.ShapeDtypeStruct((N, 1), jnp.float32),
            jax.ShapeDtypeStruct((N, 1), jnp.float32),
        ],
    )(num, den, b, gwt, asw, adw)


def _final_body(num_ref, den_ref, b3_ref, t_ref,
                wi0_ref, bl0_ref, wi1_ref, bl1_ref, wi2_ref, bl2_ref,
                t1_ref, tb1_ref, t2_ref, tb2_ref,
                l1a_ref, l1b_ref, lb1_ref, l2_ref, lb2_ref, l3_ref, lb3_ref,
                out_ref):
    ns = jnp.concatenate([num_ref[0], num_ref[1]], axis=-1)
    den = den_ref[...]
    h3 = _leaky(ns / den + b3_ref[...], 0.01)

    hh = t_ref[...]
    for wi, bl in ((wi0_ref, bl0_ref), (wi1_ref, bl1_ref), (wi2_ref, bl2_ref)):
        g = jnp.dot(hh, wi[...], preferred_element_type=jnp.float32) + bl[...]
        gi = g[:, 0:64]
        gg = g[:, 128:192]
        go = g[:, 192:256]
        c2 = jax.nn.sigmoid(gi) * jnp.tanh(gg)
        hh = jax.nn.sigmoid(go) * jnp.tanh(c2)

    t = _leaky(jnp.dot(hh, t1_ref[...], preferred_element_type=jnp.float32)
               + tb1_ref[...], 0.01)
    t = _leaky(jnp.dot(t, t2_ref[...], preferred_element_type=jnp.float32)
               + tb2_ref[...], 0.01)

    z = jax.nn.relu(jnp.dot(h3, l1a_ref[...], preferred_element_type=jnp.float32)
                    + jnp.dot(t, l1b_ref[...], preferred_element_type=jnp.float32)
                    + lb1_ref[...])
    z = jax.nn.relu(jnp.dot(z, l2_ref[...], preferred_element_type=jnp.float32)
                    + lb2_ref[...])
    out_ref[...] = jnp.dot(z, l3_ref[...], preferred_element_type=jnp.float32) + lb3_ref[...]


def _final_stage(num, den, b3, t_in, args):
    grid = (N // _R,)
    in_specs = [
        pl.BlockSpec((2, _R, 32), lambda i: (0, i, 0)),
        pl.BlockSpec((_R, 1), lambda i: (i, 0)),
        _wspec((1, 64)),
        pl.BlockSpec((_R, INPUT_LENGTH), lambda i: (i, 0)),
        _wspec((INPUT_LENGTH, 256)), _wspec((1, 256)),
        _wspec((64, 256)), _wspec((1, 256)),
        _wspec((64, 256)), _wspec((1, 256)),
        _wspec((64, 64)), _wspec((1, 64)),
        _wspec((64, 64)), _wspec((1, 64)),
        _wspec((64, 64)), _wspec((64, 64)), _wspec((1, 64)),
        _wspec((64, 64)), _wspec((1, 64)),
        _wspec((64, 4)), _wspec((1, 4)),
    ]
    return pl.pallas_call(
        _final_body,
        grid=grid,
        in_specs=in_specs,
        out_specs=pl.BlockSpec((_R, 4), lambda i: (i, 0)),
        out_shape=jax.ShapeDtypeStruct((N, 4), jnp.float32),
    )(num, den, b3, t_in, *args)


def _pad_nodes(v):
    return jnp.pad(v[:, 0], (0, NPAD - N)).reshape(NPAD // 128, 128)


def kernel(x, edge_index, params):
    p = params
    x_sec = x[:, 16:56]
    t_in = x[:, 421:421 + INPUT_LENGTH]

    loop = jnp.arange(N, dtype=jnp.int32)
    pad_e = EP - (E + N)
    src_full = jnp.concatenate(
        [edge_index[0], loop, jnp.zeros((pad_e,), jnp.int32)])
    dst_full = jnp.concatenate(
        [edge_index[1], loop, jnp.full((pad_e,), N, jnp.int32)])
    srcp = src_full.reshape(NS, CH, 128)
    dstp = dst_full.reshape(NS, CH, 128)

    row2 = lambda v: v.reshape(1, -1)
    rden = lambda d: d[0].reshape(NACC, 1)[:N]

    ha1, hb1, as1, ad1 = _pre_stage(
        x_sec, p['sec1_w'].T, row2(p['sec1_b']), p['sec2_w'].T,
        row2(p['sec2_b']), p['gat1_w'].T, row2(p['gat1_asrc']),
        row2(p['gat1_adst']))
    num1, den1 = _edge128(ha1, hb1, _pad_nodes(as1), _pad_nodes(ad1), srcp, dstp)

    ha2, hb2, as2, ad2 = _mid_stage(num1, rden(den1), row2(p['gat1_b']),
                                    p['gat2_w'].T, row2(p['gat2_asrc']),
                                    row2(p['gat2_adst']), 128, 128)
    num2, den2 = _edge128(ha2, hb2, _pad_nodes(as2), _pad_nodes(ad2), srcp, dstp)

    ha3, hb3, as3, ad3 = _mid_stage(num2, rden(den2), row2(p['gat2_b']),
                                    p['gat3_w'].T, row2(p['gat3_asrc']),
                                    row2(p['gat3_adst']), 128, 64)
    num3, den3 = _edge64(ha3, hb3, _pad_nodes(as3), _pad_nodes(ad3), srcp, dstp)

    final_args = [
        p['lstm_w_ih_0'].T, row2(p['lstm_b_ih_0'] + p['lstm_b_hh_0']),
        p['lstm_w_ih_1'].T, row2(p['lstm_b_ih_1'] + p['lstm_b_hh_1']),
        p['lstm_w_ih_2'].T, row2(p['lstm_b_ih_2'] + p['lstm_b_hh_2']),
        p['time1_w'].T, row2(p['time1_b']),
        p['time2_w'].T, row2(p['time2_b']),
        p['lin1_w'][:, :64].T, p['lin1_w'][:, 64:].T, row2(p['lin1_b']),
        p['lin2_w'].T, row2(p['lin2_b']),
        p['lin3_w'].T, row2(p['lin3_b']),
    ]
    return _final_stage(num3, rden(den3), row2(p['gat3_b']), t_in, final_args)


# speedup vs baseline: 27.3140x; 1.4278x over previous
import functools

import jax
import jax.numpy as jnp
from jax import lax
from jax.experimental import pallas as pl
from jax.experimental.pallas import tpu as pltpu
from jax.experimental.pallas import tpu_sc as plsc

N = 10000
E = 320000
INPUT_LENGTH = 24

NC = 2
NS = 16
CH = 162
EW = CH * 128
EP = NS * EW
NPAD = 10240
NACC = 10240
ROWS_PER_SUB = NACC // NS

NDR = NPAD // 128


def _make_edge_kernel(F):
    FH = F // 2
    NQ = FH // 16
    mesh = plsc.VectorSubcoreMesh(core_axis_name="c", subcore_axis_name="s")

    @functools.partial(
        pl.kernel,
        out_type=(jax.ShapeDtypeStruct((NC, NACC, FH), jnp.float32),
                  jax.ShapeDtypeStruct((NC, NDR, 128), jnp.float32)),
        mesh=mesh,
        compiler_params=pltpu.CompilerParams(needs_layout_passes=False,
                                             use_tc_tiling_on_sc=False),
        scratch_types=[
            pltpu.VMEM((CH, 128), jnp.int32),
            pltpu.VMEM((CH, 128), jnp.int32),
            pltpu.VMEM((NDR, 128), jnp.float32),
            pltpu.VMEM((NDR, 128), jnp.float32),
            pltpu.VMEM((8, 16), jnp.float32),
            pltpu.VMEM((128, FH), jnp.float32),
            pltpu.VMEM((128, FH), jnp.float32),
            pltpu.VMEM((NDR, 128), jnp.float32),
            pltpu.VMEM((1, NDR), jnp.int32),
            pltpu.VMEM_SHARED((NACC, FH), jnp.float32),
            pltpu.VMEM_SHARED((NDR, 128), jnp.float32),
            pltpu.SemaphoreType.DMA,
            pltpu.SemaphoreType.DMA,
        ],
    )
    def edge_kernel(ha_hbm, hb_hbm, as_hbm, ad_hbm, src_hbm, dst_hbm, out_hbm,
                    oden_hbm, src_v, dst_v, as_v, ad_v, ea_v, rows0_v, rows1_v,
                    denl_v, iden_v, acc_sh, den_sh, sem0, sem1):
        c = lax.axis_index("c")
        s = lax.axis_index("s")

        pltpu.sync_copy(src_hbm.at[s], src_v)
        pltpu.sync_copy(dst_hbm.at[s], dst_v)
        pltpu.sync_copy(as_hbm, as_v)
        pltpu.sync_copy(ad_hbm, ad_v)

        zv = jnp.zeros((16,), jnp.float32)

        def zden_body(r, carry):
            for q in range(8):
                denl_v[r, pl.ds(q * 16, 16)] = zv
            return carry

        lax.fori_loop(0, NDR, zden_body, 0)

        def zrow_body(r, carry):
            for q in range(NQ):
                rows0_v[r, pl.ds(q * 16, 16)] = zv
            return carry

        lax.fori_loop(0, 128, zrow_body, 0)

        for k in range(ROWS_PER_SUB // 128):
            pltpu.sync_copy(
                rows0_v, acc_sh.at[pl.ds(s * ROWS_PER_SUB + k * 128, 128)])

        @pl.when(s < NDR // 8)
        def _():
            pltpu.sync_copy(denl_v.at[pl.ds(0, 8)],
                            den_sh.at[pl.ds(s * 8, 8)])

        for i in range(NDR // 16):
            iden_v[0, pl.ds(i * 16, 16)] = (
                lax.iota(jnp.int32, 16) + jnp.int32(i * 16))
        plsc.subcore_barrier()

        def issue_gather(j, buf, sem):
            @pl.when(c == 0)
            def _():
                pltpu.async_copy(ha_hbm.at[src_v.at[j]], buf, sem)

            @pl.when(c == 1)
            def _():
                pltpu.async_copy(hb_hbm.at[src_v.at[j]], buf, sem)

        def wait_gather(j, buf, sem):
            @pl.when(c == 0)
            def _():
                pltpu.make_async_copy(ha_hbm.at[src_v.at[j]], buf, sem).wait()

            @pl.when(c == 1)
            def _():
                pltpu.make_async_copy(hb_hbm.at[src_v.at[j]], buf, sem).wait()

        def process_chunk(j, buf, sem, nbuf, nsem):
            @pl.when(j + 1 < CH)
            def _():
                issue_gather(j + 1, nbuf, nsem)

            for l in range(8):
                sv = src_v[j, pl.ds(l * 16, 16)]
                dv = dst_v[j, pl.ds(l * 16, 16)]
                a = (plsc.load_gather(as_v, [sv >> 7, sv & 127])
                     + plsc.load_gather(ad_v, [dv >> 7, dv & 127]))
                a = jnp.maximum(a, 0.2 * a)
                ea = jnp.exp(a)
                ea_v[l] = ea
                plsc.addupdate_scatter(denl_v, [dv >> 7, dv & 127], ea)

            wait_gather(j, buf, sem)

            def row_body(r, carry2):
                rv = jnp.full((16,), r, jnp.int32)
                eav = plsc.load_gather(ea_v, [rv >> 4, rv & 15])
                for q in range(NQ):
                    buf[r, pl.ds(q * 16, 16)] = buf[r, pl.ds(q * 16, 16)] * eav
                return carry2

            lax.fori_loop(0, 128, row_body, 0)
            pltpu.sync_copy(buf, acc_sh.at[dst_v.at[j]], add=True)

        issue_gather(0, rows0_v, sem0)

        def chunk_body(i, carry):
            process_chunk(2 * i, rows0_v, sem0, rows1_v, sem1)
            process_chunk(2 * i + 1, rows1_v, sem1, rows0_v, sem0)
            return carry

        lax.fori_loop(0, CH // 2, chunk_body, 0)

        pltpu.sync_copy(denl_v, den_sh.at[iden_v.at[0]], add=True)
        plsc.subcore_barrier()
        pltpu.sync_copy(acc_sh.at[pl.ds(s * ROWS_PER_SUB, ROWS_PER_SUB)],
                        out_hbm.at[c, pl.ds(s * ROWS_PER_SUB, ROWS_PER_SUB)])

        @pl.when(s < NDR // 8)
        def _():
            pltpu.sync_copy(den_sh.at[pl.ds(s * 8, 8)],
                            oden_hbm.at[c, pl.ds(s * 8, 8)])

    return edge_kernel


_edge128 = _make_edge_kernel(128)
_edge64 = _make_edge_kernel(64)


_R = 1000


def _leaky(x, slope):
    return jnp.maximum(x, slope * x)


def _wspec(shape):
    return pl.BlockSpec(shape, lambda i: (0,) * len(shape))


def _pre_body(xs_ref, w1t_ref, b1_ref, w2t_ref, b2_ref, gwt_ref, asw_ref,
              adw_ref, ha_ref, hb_ref, as_ref, ad_ref):
    s1 = jax.nn.sigmoid(jnp.dot(xs_ref[...], w1t_ref[...],
                                preferred_element_type=jnp.float32) + b1_ref[...])
    s2 = jax.nn.sigmoid(jnp.dot(s1, w2t_ref[...],
                                preferred_element_type=jnp.float32) + b2_ref[...])
    h = jnp.dot(s2, gwt_ref[...], preferred_element_type=jnp.float32)
    ha_ref[...] = h[:, :64]
    hb_ref[...] = h[:, 64:]
    as_ref[...] = (h * asw_ref[...]).sum(-1, keepdims=True)
    ad_ref[...] = (h * adw_ref[...]).sum(-1, keepdims=True)


def _pre_stage(xs, w1t, b1, w2t, b2, gwt, asw, adw):
    grid = (N // _R,)
    return pl.pallas_call(
        _pre_body,
        grid=grid,
        in_specs=[
            pl.BlockSpec((_R, 40), lambda i: (i, 0)),
            _wspec((40, 128)), _wspec((1, 128)),
            _wspec((128, 128)), _wspec((1, 128)),
            _wspec((128, 128)), _wspec((1, 128)), _wspec((1, 128)),
        ],
        out_specs=[
            pl.BlockSpec((_R, 64), lambda i: (i, 0)),
            pl.BlockSpec((_R, 64), lambda i: (i, 0)),
            pl.BlockSpec((_R, 1), lambda i: (i, 0)),
            pl.BlockSpec((_R, 1), lambda i: (i, 0)),
        ],
        out_shape=[
            jax.ShapeDtypeStruct((N, 64), jnp.float32),
            jax.ShapeDtypeStruct((N, 64), jnp.float32),
            jax.ShapeDtypeStruct((N, 1), jnp.float32),
            jax.ShapeDtypeStruct((N, 1), jnp.float32),
        ],
    )(xs, w1t, b1, w2t, b2, gwt, asw, adw)


def _make_mid_body(fh_in, f_out):
    def _mid_body(num_ref, den_ref, b_ref, gwt_ref, asw_ref, adw_ref,
                  ha_ref, hb_ref, as_ref, ad_ref):
        ns = jnp.concatenate([num_ref[0], num_ref[1]], axis=-1)
        den = den_ref[...]
        hp = _leaky(ns / den + b_ref[...], 0.01)
        h = jnp.dot(hp, gwt_ref[...], preferred_element_type=jnp.float32)
        ha_ref[...] = h[:, :f_out // 2]
        hb_ref[...] = h[:, f_out // 2:]
        as_ref[...] = (h * asw_ref[...]).sum(-1, keepdims=True)
        ad_ref[...] = (h * adw_ref[...]).sum(-1, keepdims=True)
    return _mid_body


def _mid_stage(num, den, b, gwt, asw, adw, f_in, f_out):
    grid = (N // _R,)
    fh_in = f_in // 2
    fh_out = f_out // 2
    return pl.pallas_call(
        _make_mid_body(fh_in, f_out),
        grid=grid,
        in_specs=[
            pl.BlockSpec((2, _R, fh_in), lambda i: (0, i, 0)),
            pl.BlockSpec((_R, 1), lambda i: (i, 0)),
            _wspec((1, f_in)),
            _wspec((f_in, f_out)), _wspec((1, f_out)), _wspec((1, f_out)),
        ],
        out_specs=[
            pl.BlockSpec((_R, fh_out), lambda i: (i, 0)),
            pl.BlockSpec((_R, fh_out), lambda i: (i, 0)),
            pl.BlockSpec((_R, 1), lambda i: (i, 0)),
            pl.BlockSpec((_R, 1), lambda i: (i, 0)),
        ],
        out_shape=[
            jax.ShapeDtypeStruct((N, fh_out), jnp.float32),
            jax.ShapeDtypeStruct((N, fh_out), jnp.float32),
            jax.ShapeDtypeStruct((N, 1), jnp.float32),
            jax.ShapeDtypeStruct((N, 1), jnp.float32),
        ],
    )(num, den, b, gwt, asw, adw)


def _final_body(num_ref, den_ref, b3_ref, t_ref,
                wi0_ref, bl0_ref, wi1_ref, bl1_ref, wi2_ref, bl2_ref,
                t1_ref, tb1_ref, t2_ref, tb2_ref,
                l1a_ref, l1b_ref, lb1_ref, l2_ref, lb2_ref, l3_ref, lb3_ref,
                out_ref):
    ns = jnp.concatenate([num_ref[0], num_ref[1]], axis=-1)
    den = den_ref[...]
    h3 = _leaky(ns / den + b3_ref[...], 0.01)

    hh = t_ref[...]
    for wi, bl in ((wi0_ref, bl0_ref), (wi1_ref, bl1_ref), (wi2_ref, bl2_ref)):
        g = jnp.dot(hh, wi[...], preferred_element_type=jnp.float32) + bl[...]
        gi = g[:, 0:64]
        gg = g[:, 128:192]
        go = g[:, 192:256]
        c2 = jax.nn.sigmoid(gi) * jnp.tanh(gg)
        hh = jax.nn.sigmoid(go) * jnp.tanh(c2)

    t = _leaky(jnp.dot(hh, t1_ref[...], preferred_element_type=jnp.float32)
               + tb1_ref[...], 0.01)
    t = _leaky(jnp.dot(t, t2_ref[...], preferred_element_type=jnp.float32)
               + tb2_ref[...], 0.01)

    z = jax.nn.relu(jnp.dot(h3, l1a_ref[...], preferred_element_type=jnp.float32)
                    + jnp.dot(t, l1b_ref[...], preferred_element_type=jnp.float32)
                    + lb1_ref[...])
    z = jax.nn.relu(jnp.dot(z, l2_ref[...], preferred_element_type=jnp.float32)
                    + lb2_ref[...])
    out_ref[...] = jnp.dot(z, l3_ref[...], preferred_element_type=jnp.float32) + lb3_ref[...]


def _final_stage(num, den, b3, t_in, args):
    grid = (N // _R,)
    in_specs = [
        pl.BlockSpec((2, _R, 32), lambda i: (0, i, 0)),
        pl.BlockSpec((_R, 1), lambda i: (i, 0)),
        _wspec((1, 64)),
        pl.BlockSpec((_R, INPUT_LENGTH), lambda i: (i, 0)),
        _wspec((INPUT_LENGTH, 256)), _wspec((1, 256)),
        _wspec((64, 256)), _wspec((1, 256)),
        _wspec((64, 256)), _wspec((1, 256)),
        _wspec((64, 64)), _wspec((1, 64)),
        _wspec((64, 64)), _wspec((1, 64)),
        _wspec((64, 64)), _wspec((64, 64)), _wspec((1, 64)),
        _wspec((64, 64)), _wspec((1, 64)),
        _wspec((64, 4)), _wspec((1, 4)),
    ]
    return pl.pallas_call(
        _final_body,
        grid=grid,
        in_specs=in_specs,
        out_specs=pl.BlockSpec((_R, 4), lambda i: (i, 0)),
        out_shape=jax.ShapeDtypeStruct((N, 4), jnp.float32),
    )(num, den, b3, t_in, *args)


def _pad_nodes(v):
    return jnp.pad(v[:, 0], (0, NPAD - N)).reshape(NPAD // 128, 128)


def kernel(x, edge_index, params):
    p = params
    x_sec = x[:, 16:56]
    t_in = x[:, 421:421 + INPUT_LENGTH]

    loop = jnp.arange(N, dtype=jnp.int32)
    pad_e = EP - (E + N)
    src_full = jnp.concatenate(
        [edge_index[0], loop, jnp.zeros((pad_e,), jnp.int32)])
    dst_full = jnp.concatenate(
        [edge_index[1], loop, jnp.full((pad_e,), N, jnp.int32)])
    srcp = src_full.reshape(NS, CH, 128)
    dstp = dst_full.reshape(NS, CH, 128)

    row2 = lambda v: v.reshape(1, -1)
    rden = lambda d: d[0].reshape(NACC, 1)[:N]

    ha1, hb1, as1, ad1 = _pre_stage(
        x_sec, p['sec1_w'].T, row2(p['sec1_b']), p['sec2_w'].T,
        row2(p['sec2_b']), p['gat1_w'].T, row2(p['gat1_asrc']),
        row2(p['gat1_adst']))
    num1, den1 = _edge128(ha1, hb1, _pad_nodes(as1), _pad_nodes(ad1), srcp, dstp)

    ha2, hb2, as2, ad2 = _mid_stage(num1, rden(den1), row2(p['gat1_b']),
                                    p['gat2_w'].T, row2(p['gat2_asrc']),
                                    row2(p['gat2_adst']), 128, 128)
    num2, den2 = _edge128(ha2, hb2, _pad_nodes(as2), _pad_nodes(ad2), srcp, dstp)

    ha3, hb3, as3, ad3 = _mid_stage(num2, rden(den2), row2(p['gat2_b']),
                                    p['gat3_w'].T, row2(p['gat3_asrc']),
                                    row2(p['gat3_adst']), 128, 64)
    num3, den3 = _edge64(ha3, hb3, _pad_nodes(as3), _pad_nodes(ad3), srcp, dstp)

    final_args = [
        p['lstm_w_ih_0'].T, row2(p['lstm_b_ih_0'] + p['lstm_b_hh_0']),
        p['lstm_w_ih_1'].T, row2(p['lstm_b_ih_1'] + p['lstm_b_hh_1']),
        p['lstm_w_ih_2'].T, row2(p['lstm_b_ih_2'] + p['lstm_b_hh_2']),
        p['time1_w'].T, row2(p['time1_b']),
        p['time2_w'].T, row2(p['time2_b']),
        p['lin1_w'][:, :64].T, p['lin1_w'][:, 64:].T, row2(p['lin1_b']),
        p['lin2_w'].T, row2(p['lin2_b']),
        p['lin3_w'].T, row2(p['lin3_b']),
    ]
    return _final_stage(num3, rden(den3), row2(p['gat3_b']), t_in, final_args)


# speedup vs baseline: 27.3250x; 1.0004x over previous
import functools

import jax
import jax.numpy as jnp
from jax import lax
from jax.experimental import pallas as pl
from jax.experimental.pallas import tpu as pltpu
from jax.experimental.pallas import tpu_sc as plsc

N = 10000
E = 320000
INPUT_LENGTH = 24

NC = 2
NS = 16
CH = 162
EW = CH * 128
EP = NS * EW
NPAD = 10240
NACC = 10240
ROWS_PER_SUB = NACC // NS

NDR = NPAD // 128


def _make_edge_kernel(F):
    FH = F // 2
    NQ = FH // 16
    mesh = plsc.VectorSubcoreMesh(core_axis_name="c", subcore_axis_name="s")

    @functools.partial(
        pl.kernel,
        out_type=(jax.ShapeDtypeStruct((NC, NACC, FH), jnp.float32),
                  jax.ShapeDtypeStruct((NC, NDR, 128), jnp.float32)),
        mesh=mesh,
        compiler_params=pltpu.CompilerParams(needs_layout_passes=False,
                                             use_tc_tiling_on_sc=False),
        scratch_types=[
            pltpu.VMEM((CH, 128), jnp.int32),
            pltpu.VMEM((CH, 128), jnp.int32),
            pltpu.VMEM((NDR, 128), jnp.float32),
            pltpu.VMEM((NDR, 128), jnp.float32),
            pltpu.VMEM((8, 16), jnp.float32),
            pltpu.VMEM((128, FH), jnp.float32),
            pltpu.VMEM((128, FH), jnp.float32),
            pltpu.VMEM((NDR, 128), jnp.float32),
            pltpu.VMEM((1, NDR), jnp.int32),
            pltpu.VMEM_SHARED((NACC, FH), jnp.float32),
            pltpu.VMEM_SHARED((NDR, 128), jnp.float32),
            pltpu.SemaphoreType.DMA,
            pltpu.SemaphoreType.DMA,
            pltpu.SemaphoreType.DMA,
            pltpu.SemaphoreType.DMA,
        ],
    )
    def edge_kernel(ha_hbm, hb_hbm, as_hbm, ad_hbm, src_hbm, dst_hbm, out_hbm,
                    oden_hbm, src_v, dst_v, as_v, ad_v, ea_v, rows0_v, rows1_v,
                    denl_v, iden_v, acc_sh, den_sh, sem0, sem1, ssem0, ssem1):
        c = lax.axis_index("c")
        s = lax.axis_index("s")

        pltpu.sync_copy(src_hbm.at[s], src_v)
        pltpu.sync_copy(dst_hbm.at[s], dst_v)
        pltpu.sync_copy(as_hbm, as_v)
        pltpu.sync_copy(ad_hbm, ad_v)

        zv = jnp.zeros((16,), jnp.float32)

        def zden_body(r, carry):
            for q in range(8):
                denl_v[r, pl.ds(q * 16, 16)] = zv
            return carry

        lax.fori_loop(0, NDR, zden_body, 0)

        def zrow_body(r, carry):
            for q in range(NQ):
                rows0_v[r, pl.ds(q * 16, 16)] = zv
            return carry

        lax.fori_loop(0, 128, zrow_body, 0)

        for k in range(ROWS_PER_SUB // 128):
            pltpu.sync_copy(
                rows0_v, acc_sh.at[pl.ds(s * ROWS_PER_SUB + k * 128, 128)])

        @pl.when(s < NDR // 8)
        def _():
            pltpu.sync_copy(denl_v.at[pl.ds(0, 8)],
                            den_sh.at[pl.ds(s * 8, 8)])

        for i in range(NDR // 16):
            iden_v[0, pl.ds(i * 16, 16)] = (
                lax.iota(jnp.int32, 16) + jnp.int32(i * 16))
        plsc.subcore_barrier()

        def issue_gather(j, buf, sem):
            @pl.when(c == 0)
            def _():
                pltpu.async_copy(ha_hbm.at[src_v.at[j]], buf, sem)

            @pl.when(c == 1)
            def _():
                pltpu.async_copy(hb_hbm.at[src_v.at[j]], buf, sem)

        def wait_gather(j, buf, sem):
            @pl.when(c == 0)
            def _():
                pltpu.make_async_copy(ha_hbm.at[src_v.at[j]], buf, sem).wait()

            @pl.when(c == 1)
            def _():
                pltpu.make_async_copy(hb_hbm.at[src_v.at[j]], buf, sem).wait()

        def wait_scatter(j, buf, ssem):
            pltpu.make_async_copy(buf, acc_sh.at[dst_v.at[j]], ssem).wait()

        def process_chunk(j, buf, sem, ssem, nbuf, nsem, nssem):
            @pl.when(jnp.logical_and(j >= 1, j + 1 < CH))
            def _():
                wait_scatter(j - 1, nbuf, nssem)

            @pl.when(j + 1 < CH)
            def _():
                issue_gather(j + 1, nbuf, nsem)

            for l in range(8):
                sv = src_v[j, pl.ds(l * 16, 16)]
                dv = dst_v[j, pl.ds(l * 16, 16)]
                a = (plsc.load_gather(as_v, [sv >> 7, sv & 127])
                     + plsc.load_gather(ad_v, [dv >> 7, dv & 127]))
                a = jnp.maximum(a, 0.2 * a)
                ea = jnp.exp(a)
                ea_v[l] = ea
                plsc.addupdate_scatter(denl_v, [dv >> 7, dv & 127], ea)

            wait_gather(j, buf, sem)

            def row_body(r, carry2):
                rv = jnp.full((16,), r, jnp.int32)
                eav = plsc.load_gather(ea_v, [rv >> 4, rv & 15])
                for q in range(NQ):
                    buf[r, pl.ds(q * 16, 16)] = buf[r, pl.ds(q * 16, 16)] * eav
                return carry2

            lax.fori_loop(0, 128, row_body, 0)
            pltpu.async_copy(buf, acc_sh.at[dst_v.at[j]], ssem, add=True)

        issue_gather(0, rows0_v, sem0)

        def chunk_body(i, carry):
            process_chunk(2 * i, rows0_v, sem0, ssem0, rows1_v, sem1, ssem1)
            process_chunk(2 * i + 1, rows1_v, sem1, ssem1, rows0_v, sem0, ssem0)
            return carry

        lax.fori_loop(0, CH // 2, chunk_body, 0)
        wait_scatter(CH - 2, rows0_v, ssem0)
        wait_scatter(CH - 1, rows1_v, ssem1)

        pltpu.sync_copy(denl_v, den_sh.at[iden_v.at[0]], add=True)
        plsc.subcore_barrier()
        pltpu.sync_copy(acc_sh.at[pl.ds(s * ROWS_PER_SUB, ROWS_PER_SUB)],
                        out_hbm.at[c, pl.ds(s * ROWS_PER_SUB, ROWS_PER_SUB)])

        @pl.when(s < NDR // 8)
        def _():
            pltpu.sync_copy(den_sh.at[pl.ds(s * 8, 8)],
                            oden_hbm.at[c, pl.ds(s * 8, 8)])

    return edge_kernel


_edge128 = _make_edge_kernel(128)
_edge64 = _make_edge_kernel(64)


_R = 1000


def _leaky(x, slope):
    return jnp.maximum(x, slope * x)


def _wspec(shape):
    return pl.BlockSpec(shape, lambda i: (0,) * len(shape))


def _pre_body(xs_ref, w1t_ref, b1_ref, w2t_ref, b2_ref, gwt_ref, asw_ref,
              adw_ref, ha_ref, hb_ref, as_ref, ad_ref):
    s1 = jax.nn.sigmoid(jnp.dot(xs_ref[...], w1t_ref[...],
                                preferred_element_type=jnp.float32) + b1_ref[...])
    s2 = jax.nn.sigmoid(jnp.dot(s1, w2t_ref[...],
                                preferred_element_type=jnp.float32) + b2_ref[...])
    h = jnp.dot(s2, gwt_ref[...], preferred_element_type=jnp.float32)
    ha_ref[...] = h[:, :64]
    hb_ref[...] = h[:, 64:]
    as_ref[...] = (h * asw_ref[...]).sum(-1, keepdims=True)
    ad_ref[...] = (h * adw_ref[...]).sum(-1, keepdims=True)


def _pre_stage(xs, w1t, b1, w2t, b2, gwt, asw, adw):
    grid = (N // _R,)
    return pl.pallas_call(
        _pre_body,
        grid=grid,
        in_specs=[
            pl.BlockSpec((_R, 40), lambda i: (i, 0)),
            _wspec((40, 128)), _wspec((1, 128)),
            _wspec((128, 128)), _wspec((1, 128)),
            _wspec((128, 128)), _wspec((1, 128)), _wspec((1, 128)),
        ],
        out_specs=[
            pl.BlockSpec((_R, 64), lambda i: (i, 0)),
            pl.BlockSpec((_R, 64), lambda i: (i, 0)),
            pl.BlockSpec((_R, 1), lambda i: (i, 0)),
            pl.BlockSpec((_R, 1), lambda i: (i, 0)),
        ],
        out_shape=[
            jax.ShapeDtypeStruct((N, 64), jnp.float32),
            jax.ShapeDtypeStruct((N, 64), jnp.float32),
            jax.ShapeDtypeStruct((N, 1), jnp.float32),
            jax.ShapeDtypeStruct((N, 1), jnp.float32),
        ],
    )(xs, w1t, b1, w2t, b2, gwt, asw, adw)


def _make_mid_body(fh_in, f_out):
    def _mid_body(num_ref, den_ref, b_ref, gwt_ref, asw_ref, adw_ref,
                  ha_ref, hb_ref, as_ref, ad_ref):
        ns = jnp.concatenate([num_ref[0], num_ref[1]], axis=-1)
        den = den_ref[...]
        hp = _leaky(ns / den + b_ref[...], 0.01)
        h = jnp.dot(hp, gwt_ref[...], preferred_element_type=jnp.float32)
        ha_ref[...] = h[:, :f_out // 2]
        hb_ref[...] = h[:, f_out // 2:]
        as_ref[...] = (h * asw_ref[...]).sum(-1, keepdims=True)
        ad_ref[...] = (h * adw_ref[...]).sum(-1, keepdims=True)
    return _mid_body


def _mid_stage(num, den, b, gwt, asw, adw, f_in, f_out):
    grid = (N // _R,)
    fh_in = f_in // 2
    fh_out = f_out // 2
    return pl.pallas_call(
        _make_mid_body(fh_in, f_out),
        grid=grid,
        in_specs=[
            pl.BlockSpec((2, _R, fh_in), lambda i: (0, i, 0)),
            pl.BlockSpec((_R, 1), lambda i: (i, 0)),
            _wspec((1, f_in)),
            _wspec((f_in, f_out)), _wspec((1, f_out)), _wspec((1, f_out)),
        ],
        out_specs=[
            pl.BlockSpec((_R, fh_out), lambda i: (i, 0)),
            pl.BlockSpec((_R, fh_out), lambda i: (i, 0)),
            pl.BlockSpec((_R, 1), lambda i: (i, 0)),
            pl.BlockSpec((_R, 1), lambda i: (i, 0)),
        ],
        out_shape=[
            jax.ShapeDtypeStruct((N, fh_out), jnp.float32),
            jax.ShapeDtypeStruct((N, fh_out), jnp.float32),
            jax.ShapeDtypeStruct((N, 1), jnp.float32),
            jax.ShapeDtypeStruct((N, 1), jnp.float32),
        ],
    )(num, den, b, gwt, asw, adw)


def _final_body(num_ref, den_ref, b3_ref, t_ref,
                wi0_ref, bl0_ref, wi1_ref, bl1_ref, wi2_ref, bl2_ref,
                t1_ref, tb1_ref, t2_ref, tb2_ref,
                l1a_ref, l1b_ref, lb1_ref, l2_ref, lb2_ref, l3_ref, lb3_ref,
                out_ref):
    ns = jnp.concatenate([num_ref[0], num_ref[1]], axis=-1)
    den = den_ref[...]
    h3 = _leaky(ns / den + b3_ref[...], 0.01)

    hh = t_ref[...]
    for wi, bl in ((wi0_ref, bl0_ref), (wi1_ref, bl1_ref), (wi2_ref, bl2_ref)):
        g = jnp.dot(hh, wi[...], preferred_element_type=jnp.float32) + bl[...]
        gi = g[:, 0:64]
        gg = g[:, 128:192]
        go = g[:, 192:256]
        c2 = jax.nn.sigmoid(gi) * jnp.tanh(gg)
        hh = jax.nn.sigmoid(go) * jnp.tanh(c2)

    t = _leaky(jnp.dot(hh, t1_ref[...], preferred_element_type=jnp.float32)
               + tb1_ref[...], 0.01)
    t = _leaky(jnp.dot(t, t2_ref[...], preferred_element_type=jnp.float32)
               + tb2_ref[...], 0.01)

    z = jax.nn.relu(jnp.dot(h3, l1a_ref[...], preferred_element_type=jnp.float32)
                    + jnp.dot(t, l1b_ref[...], preferred_element_type=jnp.float32)
                    + lb1_ref[...])
    z = jax.nn.relu(jnp.dot(z, l2_ref[...], preferred_element_type=jnp.float32)
                    + lb2_ref[...])
    out_ref[...] = jnp.dot(z, l3_ref[...], preferred_element_type=jnp.float32) + lb3_ref[...]


def _final_stage(num, den, b3, t_in, args):
    grid = (N // _R,)
    in_specs = [
        pl.BlockSpec((2, _R, 32), lambda i: (0, i, 0)),
        pl.BlockSpec((_R, 1), lambda i: (i, 0)),
        _wspec((1, 64)),
        pl.BlockSpec((_R, INPUT_LENGTH), lambda i: (i, 0)),
        _wspec((INPUT_LENGTH, 256)), _wspec((1, 256)),
        _wspec((64, 256)), _wspec((1, 256)),
        _wspec((64, 256)), _wspec((1, 256)),
        _wspec((64, 64)), _wspec((1, 64)),
        _wspec((64, 64)), _wspec((1, 64)),
        _wspec((64, 64)), _wspec((64, 64)), _wspec((1, 64)),
        _wspec((64, 64)), _wspec((1, 64)),
        _wspec((64, 4)), _wspec((1, 4)),
    ]
    return pl.pallas_call(
        _final_body,
        grid=grid,
        in_specs=in_specs,
        out_specs=pl.BlockSpec((_R, 4), lambda i: (i, 0)),
        out_shape=jax.ShapeDtypeStruct((N, 4), jnp.float32),
    )(num, den, b3, t_in, *args)


def _pad_nodes(v):
    return jnp.pad(v[:, 0], (0, NPAD - N)).reshape(NPAD // 128, 128)


def kernel(x, edge_index, params):
    p = params
    x_sec = x[:, 16:56]
    t_in = x[:, 421:421 + INPUT_LENGTH]

    loop = jnp.arange(N, dtype=jnp.int32)
    pad_e = EP - (E + N)
    src_full = jnp.concatenate(
        [edge_index[0], loop, jnp.zeros((pad_e,), jnp.int32)])
    dst_full = jnp.concatenate(
        [edge_index[1], loop, jnp.full((pad_e,), N, jnp.int32)])
    srcp = src_full.reshape(NS, CH, 128)
    dstp = dst_full.reshape(NS, CH, 128)

    row2 = lambda v: v.reshape(1, -1)
    rden = lambda d: d[0].reshape(NACC, 1)[:N]

    ha1, hb1, as1, ad1 = _pre_stage(
        x_sec, p['sec1_w'].T, row2(p['sec1_b']), p['sec2_w'].T,
        row2(p['sec2_b']), p['gat1_w'].T, row2(p['gat1_asrc']),
        row2(p['gat1_adst']))
    num1, den1 = _edge128(ha1, hb1, _pad_nodes(as1), _pad_nodes(ad1), srcp, dstp)

    ha2, hb2, as2, ad2 = _mid_stage(num1, rden(den1), row2(p['gat1_b']),
                                    p['gat2_w'].T, row2(p['gat2_asrc']),
                                    row2(p['gat2_adst']), 128, 128)
    num2, den2 = _edge128(ha2, hb2, _pad_nodes(as2), _pad_nodes(ad2), srcp, dstp)

    ha3, hb3, as3, ad3 = _mid_stage(num2, rden(den2), row2(p['gat2_b']),
                                    p['gat3_w'].T, row2(p['gat3_asrc']),
                                    row2(p['gat3_adst']), 128, 64)
    num3, den3 = _edge64(ha3, hb3, _pad_nodes(as3), _pad_nodes(ad3), srcp, dstp)

    final_args = [
        p['lstm_w_ih_0'].T, row2(p['lstm_b_ih_0'] + p['lstm_b_hh_0']),
        p['lstm_w_ih_1'].T, row2(p['lstm_b_ih_1'] + p['lstm_b_hh_1']),
        p['lstm_w_ih_2'].T, row2(p['lstm_b_ih_2'] + p['lstm_b_hh_2']),
        p['time1_w'].T, row2(p['time1_b']),
        p['time2_w'].T, row2(p['time2_b']),
        p['lin1_w'][:, :64].T, p['lin1_w'][:, 64:].T, row2(p['lin1_b']),
        p['lin2_w'].T, row2(p['lin2_b']),
        p['lin3_w'].T, row2(p['lin3_b']),
    ]
    return _final_stage(num3, rden(den3), row2(p['gat3_b']), t_in, final_args)


# speedup vs baseline: 28.9407x; 1.0591x over previous
import functools

import jax
import jax.numpy as jnp
from jax import lax
from jax.experimental import pallas as pl
from jax.experimental.pallas import tpu as pltpu
from jax.experimental.pallas import tpu_sc as plsc

N = 10000
E = 320000
INPUT_LENGTH = 24

NC = 2
NS = 16
CH = 162
EW = CH * 128
EP = NS * EW
NPAD = 10240
NACC = 10240
ROWS_PER_SUB = NACC // NS

NDR = NPAD // 128


def _make_edge_kernel(F):
    FH = F // 2
    NQ = FH // 16
    mesh = plsc.VectorSubcoreMesh(core_axis_name="c", subcore_axis_name="s")

    @functools.partial(
        pl.kernel,
        out_type=(jax.ShapeDtypeStruct((NC, NACC, FH), jnp.float32),
                  jax.ShapeDtypeStruct((NC, NDR, 128), jnp.float32)),
        mesh=mesh,
        compiler_params=pltpu.CompilerParams(needs_layout_passes=False,
                                             use_tc_tiling_on_sc=False),
        scratch_types=[
            pltpu.VMEM((CH, 128), jnp.int32),
            pltpu.VMEM((CH, 128), jnp.int32),
            pltpu.VMEM((NDR, 128), jnp.float32),
            pltpu.VMEM((NDR, 128), jnp.float32),
            pltpu.VMEM((8, 16), jnp.float32),
            pltpu.VMEM((128, FH), jnp.float32),
            pltpu.VMEM((128, FH), jnp.float32),
            pltpu.VMEM((NDR, 128), jnp.float32),
            pltpu.VMEM((1, NDR), jnp.int32),
            pltpu.VMEM_SHARED((NACC, FH), jnp.float32),
            pltpu.VMEM_SHARED((NDR, 128), jnp.float32),
            pltpu.SemaphoreType.DMA,
            pltpu.SemaphoreType.DMA,
            pltpu.SemaphoreType.DMA,
            pltpu.SemaphoreType.DMA,
        ],
    )
    def edge_kernel(ha_hbm, hb_hbm, as_hbm, ad_hbm, src_hbm, dst_hbm, out_hbm,
                    oden_hbm, src_v, dst_v, as_v, ad_v, ea_v, rows0_v, rows1_v,
                    denl_v, iden_v, acc_sh, den_sh, sem0, sem1, ssem0, ssem1):
        c = lax.axis_index("c")
        s = lax.axis_index("s")

        pltpu.sync_copy(src_hbm.at[s], src_v)
        pltpu.sync_copy(dst_hbm.at[s], dst_v)
        pltpu.sync_copy(as_hbm, as_v)
        pltpu.sync_copy(ad_hbm, ad_v)

        zv = jnp.zeros((16,), jnp.float32)

        def zden_body(r, carry):
            for q in range(8):
                denl_v[r, pl.ds(q * 16, 16)] = zv
            return carry

        lax.fori_loop(0, NDR, zden_body, 0)

        def zrow_body(r, carry):
            for q in range(NQ):
                rows0_v[r, pl.ds(q * 16, 16)] = zv
            return carry

        lax.fori_loop(0, 128, zrow_body, 0)

        for k in range(ROWS_PER_SUB // 128):
            pltpu.sync_copy(
                rows0_v, acc_sh.at[pl.ds(s * ROWS_PER_SUB + k * 128, 128)])

        @pl.when(s < NDR // 8)
        def _():
            pltpu.sync_copy(denl_v.at[pl.ds(0, 8)],
                            den_sh.at[pl.ds(s * 8, 8)])

        for i in range(NDR // 16):
            iden_v[0, pl.ds(i * 16, 16)] = (
                lax.iota(jnp.int32, 16) + jnp.int32(i * 16))
        plsc.subcore_barrier()

        def issue_gather(j, buf, sem):
            @pl.when(c == 0)
            def _():
                pltpu.async_copy(ha_hbm.at[src_v.at[j]], buf, sem)

            @pl.when(c == 1)
            def _():
                pltpu.async_copy(hb_hbm.at[src_v.at[j]], buf, sem)

        def wait_gather(j, buf, sem):
            @pl.when(c == 0)
            def _():
                pltpu.make_async_copy(ha_hbm.at[src_v.at[j]], buf, sem).wait()

            @pl.when(c == 1)
            def _():
                pltpu.make_async_copy(hb_hbm.at[src_v.at[j]], buf, sem).wait()

        def wait_scatter(j, buf, ssem):
            pltpu.make_async_copy(buf, acc_sh.at[dst_v.at[j]], ssem).wait()

        def process_chunk(j, buf, sem, ssem, nbuf, nsem, nssem):
            @pl.when(jnp.logical_and(j >= 1, j + 1 < CH))
            def _():
                wait_scatter(j - 1, nbuf, nssem)

            @pl.when(j + 1 < CH)
            def _():
                issue_gather(j + 1, nbuf, nsem)

            for l in range(8):
                sv = src_v[j, pl.ds(l * 16, 16)]
                dv = dst_v[j, pl.ds(l * 16, 16)]
                a = (plsc.load_gather(as_v, [sv >> 7, sv & 127])
                     + plsc.load_gather(ad_v, [dv >> 7, dv & 127]))
                a = jnp.maximum(a, 0.2 * a)
                ea = jnp.exp(a)
                ea_v[l] = ea
                plsc.addupdate_scatter(denl_v, [dv >> 7, dv & 127], ea)

            wait_gather(j, buf, sem)

            def row_body(r4, carry2):
                r0 = r4 * 4
                for u in range(4):
                    r = r0 + u
                    rv = jnp.full((16,), r, jnp.int32)
                    eav = plsc.load_gather(ea_v, [rv >> 4, rv & 15])
                    for q in range(NQ):
                        buf[r, pl.ds(q * 16, 16)] = (
                            buf[r, pl.ds(q * 16, 16)] * eav)
                return carry2

            lax.fori_loop(0, 32, row_body, 0)
            pltpu.async_copy(buf, acc_sh.at[dst_v.at[j]], ssem, add=True)

        issue_gather(0, rows0_v, sem0)

        def chunk_body(i, carry):
            process_chunk(2 * i, rows0_v, sem0, ssem0, rows1_v, sem1, ssem1)
            process_chunk(2 * i + 1, rows1_v, sem1, ssem1, rows0_v, sem0, ssem0)
            return carry

        lax.fori_loop(0, CH // 2, chunk_body, 0)
        wait_scatter(CH - 2, rows0_v, ssem0)
        wait_scatter(CH - 1, rows1_v, ssem1)

        pltpu.sync_copy(denl_v, den_sh.at[iden_v.at[0]], add=True)
        plsc.subcore_barrier()
        pltpu.sync_copy(acc_sh.at[pl.ds(s * ROWS_PER_SUB, ROWS_PER_SUB)],
                        out_hbm.at[c, pl.ds(s * ROWS_PER_SUB, ROWS_PER_SUB)])

        @pl.when(s < NDR // 8)
        def _():
            pltpu.sync_copy(den_sh.at[pl.ds(s * 8, 8)],
                            oden_hbm.at[c, pl.ds(s * 8, 8)])

    return edge_kernel


_edge128 = _make_edge_kernel(128)
_edge64 = _make_edge_kernel(64)


_R = 1000


def _leaky(x, slope):
    return jnp.maximum(x, slope * x)


def _wspec(shape):
    return pl.BlockSpec(shape, lambda i: (0,) * len(shape))


def _pre_body(xs_ref, w1t_ref, b1_ref, w2t_ref, b2_ref, gwt_ref, asw_ref,
              adw_ref, ha_ref, hb_ref, as_ref, ad_ref):
    s1 = jax.nn.sigmoid(jnp.dot(xs_ref[...], w1t_ref[...],
                                preferred_element_type=jnp.float32) + b1_ref[...])
    s2 = jax.nn.sigmoid(jnp.dot(s1, w2t_ref[...],
                                preferred_element_type=jnp.float32) + b2_ref[...])
    h = jnp.dot(s2, gwt_ref[...], preferred_element_type=jnp.float32)
    ha_ref[...] = h[:, :64]
    hb_ref[...] = h[:, 64:]
    as_ref[...] = (h * asw_ref[...]).sum(-1, keepdims=True)
    ad_ref[...] = (h * adw_ref[...]).sum(-1, keepdims=True)


def _pre_stage(xs, w1t, b1, w2t, b2, gwt, asw, adw):
    grid = (N // _R,)
    return pl.pallas_call(
        _pre_body,
        grid=grid,
        in_specs=[
            pl.BlockSpec((_R, 40), lambda i: (i, 0)),
            _wspec((40, 128)), _wspec((1, 128)),
            _wspec((128, 128)), _wspec((1, 128)),
            _wspec((128, 128)), _wspec((1, 128)), _wspec((1, 128)),
        ],
        out_specs=[
            pl.BlockSpec((_R, 64), lambda i: (i, 0)),
            pl.BlockSpec((_R, 64), lambda i: (i, 0)),
            pl.BlockSpec((_R, 1), lambda i: (i, 0)),
            pl.BlockSpec((_R, 1), lambda i: (i, 0)),
        ],
        out_shape=[
            jax.ShapeDtypeStruct((N, 64), jnp.float32),
            jax.ShapeDtypeStruct((N, 64), jnp.float32),
            jax.ShapeDtypeStruct((N, 1), jnp.float32),
            jax.ShapeDtypeStruct((N, 1), jnp.float32),
        ],
    )(xs, w1t, b1, w2t, b2, gwt, asw, adw)


def _make_mid_body(fh_in, f_out):
    def _mid_body(num_ref, den_ref, b_ref, gwt_ref, asw_ref, adw_ref,
                  ha_ref, hb_ref, as_ref, ad_ref):
        ns = jnp.concatenate([num_ref[0], num_ref[1]], axis=-1)
        den = den_ref[...]
        hp = _leaky(ns / den + b_ref[...], 0.01)
        h = jnp.dot(hp, gwt_ref[...], preferred_element_type=jnp.float32)
        ha_ref[...] = h[:, :f_out // 2]
        hb_ref[...] = h[:, f_out // 2:]
        as_ref[...] = (h * asw_ref[...]).sum(-1, keepdims=True)
        ad_ref[...] = (h * adw_ref[...]).sum(-1, keepdims=True)
    return _mid_body


def _mid_stage(num, den, b, gwt, asw, adw, f_in, f_out):
    grid = (N // _R,)
    fh_in = f_in // 2
    fh_out = f_out // 2
    return pl.pallas_call(
        _make_mid_body(fh_in, f_out),
        grid=grid,
        in_specs=[
            pl.BlockSpec((2, _R, fh_in), lambda i: (0, i, 0)),
            pl.BlockSpec((_R, 1), lambda i: (i, 0)),
            _wspec((1, f_in)),
            _wspec((f_in, f_out)), _wspec((1, f_out)), _wspec((1, f_out)),
        ],
        out_specs=[
            pl.BlockSpec((_R, fh_out), lambda i: (i, 0)),
            pl.BlockSpec((_R, fh_out), lambda i: (i, 0)),
            pl.BlockSpec((_R, 1), lambda i: (i, 0)),
            pl.BlockSpec((_R, 1), lambda i: (i, 0)),
        ],
        out_shape=[
            jax.ShapeDtypeStruct((N, fh_out), jnp.float32),
            jax.ShapeDtypeStruct((N, fh_out), jnp.float32),
            jax.ShapeDtypeStruct((N, 1), jnp.float32),
            jax.ShapeDtypeStruct((N, 1), jnp.float32),
        ],
    )(num, den, b, gwt, asw, adw)


def _final_body(num_ref, den_ref, b3_ref, t_ref,
                wi0_ref, bl0_ref, wi1_ref, bl1_ref, wi2_ref, bl2_ref,
                t1_ref, tb1_ref, t2_ref, tb2_ref,
                l1a_ref, l1b_ref, lb1_ref, l2_ref, lb2_ref, l3_ref, lb3_ref,
                out_ref):
    ns = jnp.concatenate([num_ref[0], num_ref[1]], axis=-1)
    den = den_ref[...]
    h3 = _leaky(ns / den + b3_ref[...], 0.01)

    hh = t_ref[...]
    for wi, bl in ((wi0_ref, bl0_ref), (wi1_ref, bl1_ref), (wi2_ref, bl2_ref)):
        g = jnp.dot(hh, wi[...], preferred_element_type=jnp.float32) + bl[...]
        gi = g[:, 0:64]
        gg = g[:, 128:192]
        go = g[:, 192:256]
        c2 = jax.nn.sigmoid(gi) * jnp.tanh(gg)
        hh = jax.nn.sigmoid(go) * jnp.tanh(c2)

    t = _leaky(jnp.dot(hh, t1_ref[...], preferred_element_type=jnp.float32)
               + tb1_ref[...], 0.01)
    t = _leaky(jnp.dot(t, t2_ref[...], preferred_element_type=jnp.float32)
               + tb2_ref[...], 0.01)

    z = jax.nn.relu(jnp.dot(h3, l1a_ref[...], preferred_element_type=jnp.float32)
                    + jnp.dot(t, l1b_ref[...], preferred_element_type=jnp.float32)
                    + lb1_ref[...])
    z = jax.nn.relu(jnp.dot(z, l2_ref[...], preferred_element_type=jnp.float32)
                    + lb2_ref[...])
    out_ref[...] = jnp.dot(z, l3_ref[...], preferred_element_type=jnp.float32) + lb3_ref[...]


def _final_stage(num, den, b3, t_in, args):
    grid = (N // _R,)
    in_specs = [
        pl.BlockSpec((2, _R, 32), lambda i: (0, i, 0)),
        pl.BlockSpec((_R, 1), lambda i: (i, 0)),
        _wspec((1, 64)),
        pl.BlockSpec((_R, INPUT_LENGTH), lambda i: (i, 0)),
        _wspec((INPUT_LENGTH, 256)), _wspec((1, 256)),
        _wspec((64, 256)), _wspec((1, 256)),
        _wspec((64, 256)), _wspec((1, 256)),
        _wspec((64, 64)), _wspec((1, 64)),
        _wspec((64, 64)), _wspec((1, 64)),
        _wspec((64, 64)), _wspec((64, 64)), _wspec((1, 64)),
        _wspec((64, 64)), _wspec((1, 64)),
        _wspec((64, 4)), _wspec((1, 4)),
    ]
    return pl.pallas_call(
        _final_body,
        grid=grid,
        in_specs=in_specs,
        out_specs=pl.BlockSpec((_R, 4), lambda i: (i, 0)),
        out_shape=jax.ShapeDtypeStruct((N, 4), jnp.float32),
    )(num, den, b3, t_in, *args)


def _pad_nodes(v):
    return jnp.pad(v[:, 0], (0, NPAD - N)).reshape(NPAD // 128, 128)


def kernel(x, edge_index, params):
    p = params
    x_sec = x[:, 16:56]
    t_in = x[:, 421:421 + INPUT_LENGTH]

    loop = jnp.arange(N, dtype=jnp.int32)
    pad_e = EP - (E + N)
    src_full = jnp.concatenate(
        [edge_index[0], loop, jnp.zeros((pad_e,), jnp.int32)])
    dst_full = jnp.concatenate(
        [edge_index[1], loop, jnp.full((pad_e,), N, jnp.int32)])
    srcp = src_full.reshape(NS, CH, 128)
    dstp = dst_full.reshape(NS, CH, 128)

    row2 = lambda v: v.reshape(1, -1)
    rden = lambda d: d[0].reshape(NACC, 1)[:N]

    ha1, hb1, as1, ad1 = _pre_stage(
        x_sec, p['sec1_w'].T, row2(p['sec1_b']), p['sec2_w'].T,
        row2(p['sec2_b']), p['gat1_w'].T, row2(p['gat1_asrc']),
        row2(p['gat1_adst']))
    num1, den1 = _edge128(ha1, hb1, _pad_nodes(as1), _pad_nodes(ad1), srcp, dstp)

    ha2, hb2, as2, ad2 = _mid_stage(num1, rden(den1), row2(p['gat1_b']),
                                    p['gat2_w'].T, row2(p['gat2_asrc']),
                                    row2(p['gat2_adst']), 128, 128)
    num2, den2 = _edge128(ha2, hb2, _pad_nodes(as2), _pad_nodes(ad2), srcp, dstp)

    ha3, hb3, as3, ad3 = _mid_stage(num2, rden(den2), row2(p['gat2_b']),
                                    p['gat3_w'].T, row2(p['gat3_asrc']),
                                    row2(p['gat3_adst']), 128, 64)
    num3, den3 = _edge64(ha3, hb3, _pad_nodes(as3), _pad_nodes(ad3), srcp, dstp)

    final_args = [
        p['lstm_w_ih_0'].T, row2(p['lstm_b_ih_0'] + p['lstm_b_hh_0']),
        p['lstm_w_ih_1'].T, row2(p['lstm_b_ih_1'] + p['lstm_b_hh_1']),
        p['lstm_w_ih_2'].T, row2(p['lstm_b_ih_2'] + p['lstm_b_hh_2']),
        p['time1_w'].T, row2(p['time1_b']),
        p['time2_w'].T, row2(p['time2_b']),
        p['lin1_w'][:, :64].T, p['lin1_w'][:, 64:].T, row2(p['lin1_b']),
        p['lin2_w'].T, row2(p['lin2_b']),
        p['lin3_w'].T, row2(p['lin3_b']),
    ]
    return _final_stage(num3, rden(den3), row2(p['gat3_b']), t_in, final_args)


# speedup vs baseline: 30.9193x; 1.0684x over previous
import functools

import jax
import jax.numpy as jnp
from jax import lax
from jax.experimental import pallas as pl
from jax.experimental.pallas import tpu as pltpu
from jax.experimental.pallas import tpu_sc as plsc

N = 10000
E = 320000
INPUT_LENGTH = 24

NC = 2
NS = 16
CH = 162
EW = CH * 128
EP = NS * EW
NPAD = 10240
NACC = 10240
ROWS_PER_SUB = NACC // NS

NDR = NPAD // 128


def _make_edge_kernel(F):
    FH = F // 2
    NQ = FH // 16
    mesh = plsc.VectorSubcoreMesh(core_axis_name="c", subcore_axis_name="s")

    @functools.partial(
        pl.kernel,
        out_type=(jax.ShapeDtypeStruct((NC, NACC, FH), jnp.float32),
                  jax.ShapeDtypeStruct((NC, NDR, 128), jnp.float32)),
        mesh=mesh,
        compiler_params=pltpu.CompilerParams(needs_layout_passes=False,
                                             use_tc_tiling_on_sc=False),
        scratch_types=[
            pltpu.VMEM((CH, 128), jnp.int32),
            pltpu.VMEM((3, 128), jnp.int32),
            pltpu.VMEM((3, 128), jnp.int32),
            pltpu.VMEM((NDR, 128), jnp.float32),
            pltpu.VMEM((NDR, 128), jnp.float32),
            pltpu.VMEM((8, 16), jnp.float32),
            pltpu.VMEM((128, FH), jnp.float32),
            pltpu.VMEM((128, FH), jnp.float32),
            pltpu.VMEM((128, FH), jnp.float32),
            pltpu.VMEM((NDR, 128), jnp.float32),
            pltpu.VMEM((1, NDR), jnp.int32),
            pltpu.VMEM_SHARED((NACC, FH), jnp.float32),
            pltpu.VMEM_SHARED((NDR, 128), jnp.float32),
            pltpu.SemaphoreType.DMA,
            pltpu.SemaphoreType.DMA,
            pltpu.SemaphoreType.DMA,
            pltpu.SemaphoreType.DMA,
            pltpu.SemaphoreType.DMA,
            pltpu.SemaphoreType.DMA,
        ],
    )
    def edge_kernel(ha_hbm, hb_hbm, as_hbm, ad_hbm, pk_hbm, out_hbm,
                    oden_hbm, pk_v, srow_v, drow_v, as_v, ad_v, ea_v,
                    rows0_v, rows1_v, rows2_v, denl_v, iden_v, acc_sh, den_sh,
                    sem0, sem1, sem2, ssem0, ssem1, ssem2):
        c = lax.axis_index("c")
        s = lax.axis_index("s")

        bufs = (rows0_v, rows1_v, rows2_v)
        sems = (sem0, sem1, sem2)
        ssems = (ssem0, ssem1, ssem2)

        pltpu.sync_copy(pk_hbm.at[s], pk_v)
        pltpu.sync_copy(as_hbm, as_v)
        pltpu.sync_copy(ad_hbm, ad_v)

        zv = jnp.zeros((16,), jnp.float32)

        def zden_body(r, carry):
            for q in range(8):
                denl_v[r, pl.ds(q * 16, 16)] = zv
            return carry

        lax.fori_loop(0, NDR, zden_body, 0)

        def zrow_body(r, carry):
            for q in range(NQ):
                rows0_v[r, pl.ds(q * 16, 16)] = zv
            return carry

        lax.fori_loop(0, 128, zrow_body, 0)

        for k in range(ROWS_PER_SUB // 128):
            pltpu.sync_copy(
                rows0_v, acc_sh.at[pl.ds(s * ROWS_PER_SUB + k * 128, 128)])

        @pl.when(s < NDR // 8)
        def _():
            pltpu.sync_copy(denl_v.at[pl.ds(0, 8)],
                            den_sh.at[pl.ds(s * 8, 8)])

        for i in range(NDR // 16):
            iden_v[0, pl.ds(i * 16, 16)] = (
                lax.iota(jnp.int32, 16) + jnp.int32(i * 16))
        plsc.subcore_barrier()

        def unpack(j, k):
            for l in range(8):
                p = pk_v[j, pl.ds(l * 16, 16)]
                srow_v[k, pl.ds(l * 16, 16)] = p & 16383
                drow_v[k, pl.ds(l * 16, 16)] = p >> 14

        def issue_gather(k):
            @pl.when(c == 0)
            def _():
                pltpu.async_copy(ha_hbm.at[srow_v.at[k]], bufs[k], sems[k])

            @pl.when(c == 1)
            def _():
                pltpu.async_copy(hb_hbm.at[srow_v.at[k]], bufs[k], sems[k])

        def wait_gather(k):
            @pl.when(c == 0)
            def _():
                pltpu.make_async_copy(ha_hbm.at[srow_v.at[k]], bufs[k],
                                      sems[k]).wait()

            @pl.when(c == 1)
            def _():
                pltpu.make_async_copy(hb_hbm.at[srow_v.at[k]], bufs[k],
                                      sems[k]).wait()

        def wait_scatter(k):
            pltpu.make_async_copy(bufs[k], acc_sh.at[drow_v.at[k]],
                                  ssems[k]).wait()

        def process_chunk(j, k):
            buf = bufs[k]
            for l in range(8):
                p = pk_v[j, pl.ds(l * 16, 16)]
                sv = p & 16383
                dv = p >> 14
                a = (plsc.load_gather(as_v, [sv >> 7, sv & 127])
                     + plsc.load_gather(ad_v, [dv >> 7, dv & 127]))
                a = jnp.maximum(a, 0.2 * a)
                ea = jnp.exp(a)
                ea_v[l] = ea
                plsc.addupdate_scatter(denl_v, [dv >> 7, dv & 127], ea)

            kp = (k + 2) % 3

            @pl.when(j >= 1)
            def _():
                wait_scatter(kp)

            @pl.when(j + 2 < CH)
            def _():
                unpack(j + 2, kp)
                issue_gather(kp)

            wait_gather(k)

            def row_body(r4, carry2):
                r0 = r4 * 4
                for u in range(4):
                    r = r0 + u
                    rv = jnp.full((16,), r, jnp.int32)
                    eav = plsc.load_gather(ea_v, [rv >> 4, rv & 15])
                    for q in range(NQ):
                        buf[r, pl.ds(q * 16, 16)] = (
                            buf[r, pl.ds(q * 16, 16)] * eav)
                return carry2

            lax.fori_loop(0, 32, row_body, 0)
            pltpu.async_copy(buf, acc_sh.at[drow_v.at[k]], ssems[k], add=True)

        unpack(0, 0)
        issue_gather(0)
        unpack(1, 1)
        issue_gather(1)

        def chunk_body(i, carry):
            process_chunk(3 * i, 0)
            process_chunk(3 * i + 1, 1)
            process_chunk(3 * i + 2, 2)
            return carry

        lax.fori_loop(0, CH // 3, chunk_body, 0)
        wait_scatter((CH - 1) % 3)

        pltpu.sync_copy(denl_v, den_sh.at[iden_v.at[0]], add=True)
        plsc.subcore_barrier()
        pltpu.sync_copy(acc_sh.at[pl.ds(s * ROWS_PER_SUB, ROWS_PER_SUB)],
                        out_hbm.at[c, pl.ds(s * ROWS_PER_SUB, ROWS_PER_SUB)])

        @pl.when(s < NDR // 8)
        def _():
            pltpu.sync_copy(den_sh.at[pl.ds(s * 8, 8)],
                            oden_hbm.at[c, pl.ds(s * 8, 8)])

    return edge_kernel


_edge128 = _make_edge_kernel(128)
_edge64 = _make_edge_kernel(64)


_R = 1000


def _leaky(x, slope):
    return jnp.maximum(x, slope * x)


def _wspec(shape):
    return pl.BlockSpec(shape, lambda i: (0,) * len(shape))


def _pre_body(xs_ref, w1t_ref, b1_ref, w2t_ref, b2_ref, gwt_ref, asw_ref,
              adw_ref, ha_ref, hb_ref, as_ref, ad_ref):
    s1 = jax.nn.sigmoid(jnp.dot(xs_ref[...], w1t_ref[...],
                                preferred_element_type=jnp.float32) + b1_ref[...])
    s2 = jax.nn.sigmoid(jnp.dot(s1, w2t_ref[...],
                                preferred_element_type=jnp.float32) + b2_ref[...])
    h = jnp.dot(s2, gwt_ref[...], preferred_element_type=jnp.float32)
    ha_ref[...] = h[:, :64]
    hb_ref[...] = h[:, 64:]
    as_ref[...] = (h * asw_ref[...]).sum(-1, keepdims=True)
    ad_ref[...] = (h * adw_ref[...]).sum(-1, keepdims=True)


def _pre_stage(xs, w1t, b1, w2t, b2, gwt, asw, adw):
    grid = (N // _R,)
    return pl.pallas_call(
        _pre_body,
        grid=grid,
        in_specs=[
            pl.BlockSpec((_R, 40), lambda i: (i, 0)),
            _wspec((40, 128)), _wspec((1, 128)),
            _wspec((128, 128)), _wspec((1, 128)),
            _wspec((128, 128)), _wspec((1, 128)), _wspec((1, 128)),
        ],
        out_specs=[
            pl.BlockSpec((_R, 64), lambda i: (i, 0)),
            pl.BlockSpec((_R, 64), lambda i: (i, 0)),
            pl.BlockSpec((_R, 1), lambda i: (i, 0)),
            pl.BlockSpec((_R, 1), lambda i: (i, 0)),
        ],
        out_shape=[
            jax.ShapeDtypeStruct((N, 64), jnp.float32),
            jax.ShapeDtypeStruct((N, 64), jnp.float32),
            jax.ShapeDtypeStruct((N, 1), jnp.float32),
            jax.ShapeDtypeStruct((N, 1), jnp.float32),
        ],
    )(xs, w1t, b1, w2t, b2, gwt, asw, adw)


def _make_mid_body(fh_in, f_out):
    def _mid_body(num_ref, den_ref, b_ref, gwt_ref, asw_ref, adw_ref,
                  ha_ref, hb_ref, as_ref, ad_ref):
        ns = jnp.concatenate([num_ref[0], num_ref[1]], axis=-1)
        den = den_ref[...]
        hp = _leaky(ns / den + b_ref[...], 0.01)
        h = jnp.dot(hp, gwt_ref[...], preferred_element_type=jnp.float32)
        ha_ref[...] = h[:, :f_out // 2]
        hb_ref[...] = h[:, f_out // 2:]
        as_ref[...] = (h * asw_ref[...]).sum(-1, keepdims=True)
        ad_ref[...] = (h * adw_ref[...]).sum(-1, keepdims=True)
    return _mid_body


def _mid_stage(num, den, b, gwt, asw, adw, f_in, f_out):
    grid = (N // _R,)
    fh_in = f_in // 2
    fh_out = f_out // 2
    return pl.pallas_call(
        _make_mid_body(fh_in, f_out),
        grid=grid,
        in_specs=[
            pl.BlockSpec((2, _R, fh_in), lambda i: (0, i, 0)),
            pl.BlockSpec((_R, 1), lambda i: (i, 0)),
            _wspec((1, f_in)),
            _wspec((f_in, f_out)), _wspec((1, f_out)), _wspec((1, f_out)),
        ],
        out_specs=[
            pl.BlockSpec((_R, fh_out), lambda i: (i, 0)),
            pl.BlockSpec((_R, fh_out), lambda i: (i, 0)),
            pl.BlockSpec((_R, 1), lambda i: (i, 0)),
            pl.BlockSpec((_R, 1), lambda i: (i, 0)),
        ],
        out_shape=[
            jax.ShapeDtypeStruct((N, fh_out), jnp.float32),
            jax.ShapeDtypeStruct((N, fh_out), jnp.float32),
            jax.ShapeDtypeStruct((N, 1), jnp.float32),
            jax.ShapeDtypeStruct((N, 1), jnp.float32),
        ],
    )(num, den, b, gwt, asw, adw)


def _final_body(num_ref, den_ref, b3_ref, t_ref,
                wi0_ref, bl0_ref, wi1_ref, bl1_ref, wi2_ref, bl2_ref,
                t1_ref, tb1_ref, t2_ref, tb2_ref,
                l1a_ref, l1b_ref, lb1_ref, l2_ref, lb2_ref, l3_ref, lb3_ref,
                out_ref):
    ns = jnp.concatenate([num_ref[0], num_ref[1]], axis=-1)
    den = den_ref[...]
    h3 = _leaky(ns / den + b3_ref[...], 0.01)

    hh = t_ref[...]
    for wi, bl in ((wi0_ref, bl0_ref), (wi1_ref, bl1_ref), (wi2_ref, bl2_ref)):
        g = jnp.dot(hh, wi[...], preferred_element_type=jnp.float32) + bl[...]
        gi = g[:, 0:64]
        gg = g[:, 128:192]
        go = g[:, 192:256]
        c2 = jax.nn.sigmoid(gi) * jnp.tanh(gg)
        hh = jax.nn.sigmoid(go) * jnp.tanh(c2)

    t = _leaky(jnp.dot(hh, t1_ref[...], preferred_element_type=jnp.float32)
               + tb1_ref[...], 0.01)
    t = _leaky(jnp.dot(t, t2_ref[...], preferred_element_type=jnp.float32)
               + tb2_ref[...], 0.01)

    z = jax.nn.relu(jnp.dot(h3, l1a_ref[...], preferred_element_type=jnp.float32)
                    + jnp.dot(t, l1b_ref[...], preferred_element_type=jnp.float32)
                    + lb1_ref[...])
    z = jax.nn.relu(jnp.dot(z, l2_ref[...], preferred_element_type=jnp.float32)
                    + lb2_ref[...])
    out_ref[...] = jnp.dot(z, l3_ref[...], preferred_element_type=jnp.float32) + lb3_ref[...]


def _final_stage(num, den, b3, t_in, args):
    grid = (N // _R,)
    in_specs = [
        pl.BlockSpec((2, _R, 32), lambda i: (0, i, 0)),
        pl.BlockSpec((_R, 1), lambda i: (i, 0)),
        _wspec((1, 64)),
        pl.BlockSpec((_R, INPUT_LENGTH), lambda i: (i, 0)),
        _wspec((INPUT_LENGTH, 256)), _wspec((1, 256)),
        _wspec((64, 256)), _wspec((1, 256)),
        _wspec((64, 256)), _wspec((1, 256)),
        _wspec((64, 64)), _wspec((1, 64)),
        _wspec((64, 64)), _wspec((1, 64)),
        _wspec((64, 64)), _wspec((64, 64)), _wspec((1, 64)),
        _wspec((64, 64)), _wspec((1, 64)),
        _wspec((64, 4)), _wspec((1, 4)),
    ]
    return pl.pallas_call(
        _final_body,
        grid=grid,
        in_specs=in_specs,
        out_specs=pl.BlockSpec((_R, 4), lambda i: (i, 0)),
        out_shape=jax.ShapeDtypeStruct((N, 4), jnp.float32),
    )(num, den, b3, t_in, *args)


def _pad_nodes(v):
    return jnp.pad(v[:, 0], (0, NPAD - N)).reshape(NPAD // 128, 128)


def kernel(x, edge_index, params):
    p = params
    x_sec = x[:, 16:56]
    t_in = x[:, 421:421 + INPUT_LENGTH]

    loop = jnp.arange(N, dtype=jnp.int32)
    pad_e = EP - (E + N)
    src_full = jnp.concatenate(
        [edge_index[0].astype(jnp.int32), loop, jnp.zeros((pad_e,), jnp.int32)])
    dst_full = jnp.concatenate(
        [edge_index[1].astype(jnp.int32), loop, jnp.full((pad_e,), N, jnp.int32)])
    pkp = (src_full | (dst_full << 14)).reshape(NS, CH, 128)

    row2 = lambda v: v.reshape(1, -1)
    rden = lambda d: d[0].reshape(NACC, 1)[:N]

    ha1, hb1, as1, ad1 = _pre_stage(
        x_sec, p['sec1_w'].T, row2(p['sec1_b']), p['sec2_w'].T,
        row2(p['sec2_b']), p['gat1_w'].T, row2(p['gat1_asrc']),
        row2(p['gat1_adst']))
    num1, den1 = _edge128(ha1, hb1, _pad_nodes(as1), _pad_nodes(ad1), pkp)

    ha2, hb2, as2, ad2 = _mid_stage(num1, rden(den1), row2(p['gat1_b']),
                                    p['gat2_w'].T, row2(p['gat2_asrc']),
                                    row2(p['gat2_adst']), 128, 128)
    num2, den2 = _edge128(ha2, hb2, _pad_nodes(as2), _pad_nodes(ad2), pkp)

    ha3, hb3, as3, ad3 = _mid_stage(num2, rden(den2), row2(p['gat2_b']),
                                    p['gat3_w'].T, row2(p['gat3_asrc']),
                                    row2(p['gat3_adst']), 128, 64)
    num3, den3 = _edge64(ha3, hb3, _pad_nodes(as3), _pad_nodes(ad3), pkp)

    final_args = [
        p['lstm_w_ih_0'].T, row2(p['lstm_b_ih_0'] + p['lstm_b_hh_0']),
        p['lstm_w_ih_1'].T, row2(p['lstm_b_ih_1'] + p['lstm_b_hh_1']),
        p['lstm_w_ih_2'].T, row2(p['lstm_b_ih_2'] + p['lstm_b_hh_2']),
        p['time1_w'].T, row2(p['time1_b']),
        p['time2_w'].T, row2(p['time2_b']),
        p['lin1_w'][:, :64].T, p['lin1_w'][:, 64:].T, row2(p['lin1_b']),
        p['lin2_w'].T, row2(p['lin2_b']),
        p['lin3_w'].T, row2(p['lin3_b']),
    ]
    return _final_stage(num3, rden(den3), row2(p['gat3_b']), t_in, final_args)
